# baseline (speedup 1.0000x reference)
import functools
import jax
import jax.numpy as jnp
from jax import lax
from jax.experimental import pallas as pl
from jax.experimental.pallas import tpu as pltpu
from jax.experimental.pallas import tpu_sc as plsc

ARITY = 8
G = 256
P = 256
D = 16

NC = 2
NS = 16
NW = NC * NS

R = 8
NIDX = R * G


def _idx_kernel(x_ref, prow_ref, min_ref):
    x0 = x_ref[:, 0, :]
    absmin = jnp.abs(x0)
    ind = (x0 >= 0).astype(jnp.int32)
    for j in range(1, ARITY):
        xj = x_ref[:, j, :]
        absmin = jnp.minimum(absmin, jnp.abs(xj))
        ind = ind + (xj >= 0).astype(jnp.int32) * (2 ** j)
    g_base = jax.lax.broadcasted_iota(jnp.int32, ind.shape, 1) * P
    prow_ref[...] = g_base + ind
    min_ref[...] = absmin


def _sc_body(prow_hbm, min_hbm, p_hbm, o_hbm,
             idx_a, idx_b, min_a, min_b, gath_a, gath_b,
             semi_a, semi_b, semg_a, semg_b):
    wid = lax.axis_index("s") * NC + lax.axis_index("c")
    iota = lax.iota(jnp.int32, 16)

    nchunk = (o_hbm.shape[0] // G) // (NW * R)
    base_row = wid * nchunk * R

    def start_in(c, idx_buf, min_buf, semi):
        off = (base_row + c * R) * G
        pltpu.async_copy(prow_hbm.at[pl.ds(off, NIDX)], idx_buf, semi)
        pltpu.async_copy(min_hbm.at[pl.ds(off, NIDX)], min_buf, semi)

    def s1(idx_buf, min_buf, gath_buf, semi, semg):
        pltpu.make_async_copy(prow_hbm.at[pl.ds(0, NIDX)], idx_buf,
                              semi).wait()
        pltpu.make_async_copy(min_hbm.at[pl.ds(0, NIDX)], min_buf,
                              semi).wait()
        pltpu.async_copy(p_hbm.at[idx_buf], gath_buf, semg)

    def s2(c, idx_buf, gath_buf, min_buf, semg):
        pltpu.make_async_copy(p_hbm.at[idx_buf], gath_buf, semg).wait()

        @plsc.parallel_loop(0, NIDX // 16, unroll=2)
        def _scale(b):
            minvec = min_buf[pl.ds(b * 16, 16)]
            rbase = b * 16 + iota
            for d in range(D):
                dvec = jnp.full((16,), d, jnp.int32)
                vals = plsc.load_gather(gath_buf, [rbase, dvec])
                plsc.store_scatter(gath_buf, [rbase, dvec], vals * minvec)

        pltpu.sync_copy(gath_buf,
                        o_hbm.at[pl.ds((base_row + c * R) * G, NIDX)])

    start_in(0, idx_a, min_a, semi_a)

    @pl.loop(0, nchunk // 2)
    def _pair(k):
        c0 = 2 * k
        s1(idx_a, min_a, gath_a, semi_a, semg_a)
        start_in(c0 + 1, idx_b, min_b, semi_b)
        s2(c0, idx_a, gath_a, min_a, semg_a)
        s1(idx_b, min_b, gath_b, semi_b, semg_b)

        @pl.when(c0 + 2 < nchunk)
        def _():
            start_in(c0 + 2, idx_a, min_a, semi_a)

        s2(c0 + 1, idx_b, gath_b, min_b, semg_b)


@jax.jit
def kernel(X, params):
    B = X.shape[0]
    Bblk = 256
    Xt = X.reshape(B, G, ARITY).transpose(0, 2, 1)
    prow, minv = pl.pallas_call(
        _idx_kernel,
        grid=(B // Bblk,),
        in_specs=[pl.BlockSpec((Bblk, ARITY, G), lambda i: (i, 0, 0))],
        out_specs=[
            pl.BlockSpec((Bblk, G), lambda i: (i, 0)),
            pl.BlockSpec((Bblk, G), lambda i: (i, 0)),
        ],
        out_shape=[
            jax.ShapeDtypeStruct((B, G), jnp.int32),
            jax.ShapeDtypeStruct((B, G), jnp.float32),
        ],
    )(Xt)

    Pf = params.reshape(G * P, D)
    mesh = plsc.VectorSubcoreMesh(core_axis_name="c", subcore_axis_name="s")
    run = functools.partial(
        pl.kernel,
        out_type=jax.ShapeDtypeStruct((B * G, D), jnp.float32),
        mesh=mesh,
        compiler_params=pltpu.CompilerParams(
            needs_layout_passes=False, use_tc_tiling_on_sc=False),
        scratch_types=[
            pltpu.VMEM((NIDX,), jnp.int32),
            pltpu.VMEM((NIDX,), jnp.int32),
            pltpu.VMEM((NIDX,), jnp.float32),
            pltpu.VMEM((NIDX,), jnp.float32),
            pltpu.VMEM((NIDX, D), jnp.float32),
            pltpu.VMEM((NIDX, D), jnp.float32),
            pltpu.SemaphoreType.DMA,
            pltpu.SemaphoreType.DMA,
            pltpu.SemaphoreType.DMA,
            pltpu.SemaphoreType.DMA,
        ],
    )(_sc_body)
    out = run(prow.reshape(B * G), minv.reshape(B * G), Pf)
    return out.reshape(B, G * D)
